# baseline (speedup 1.0000x reference)
import jax
import jax.numpy as jnp
from jax.experimental import pallas as pl
from jax.experimental.pallas import tpu as pltpu

N_Q = 8
N_GROUPS = 2
N_CODES = 1024
VQ_DIM = 512
CODE_DIM = VQ_DIM // N_GROUPS
TB = 512
N_TOK = 16 * 2048
GRID = N_TOK // TB


def _rvq_body(xt_ref, cb_ref, wn_ref, qout_ref, idx_ref, ss_ref,
              wb0_ref, wb1_ref):
    step = pl.program_id(0)

    @pl.when(step == 0)
    def _split():
        cb = cb_ref[...]
        b0 = cb.astype(jnp.bfloat16)
        wb0_ref[...] = b0
        wb1_ref[...] = (cb - b0.astype(jnp.float32)).astype(jnp.bfloat16)

    x = xt_ref[...]
    r = x
    qsum = jnp.zeros_like(x)
    ss_list = []
    for i in range(N_Q):
        zparts = []
        for g in range(N_GROUPS):
            rg = r[:, g * CODE_DIM:(g + 1) * CODE_DIM]
            t = rg * rg
            v = t[:, :128] + t[:, 128:]
            acc = v[:, 0:8]
            for j in range(1, 16):
                acc = acc + v[:, 8 * j:8 * j + 8]
            f4 = acc[:, 0:4] + acc[:, 4:8]
            f2 = f4[:, 0:2] + f4[:, 2:4]
            a = f2[:, 0:1] + f2[:, 1:2]
            w = cb_ref[i, g]
            m = jax.lax.dot_general(
                rg, w, (((1,), (1,)), ((), ())),
                precision=jax.lax.Precision.DEFAULT)
            b = wn_ref[i * N_GROUPS + g, :][None, :]
            d = (a + b) - 2.0 * m
            dmin = jnp.min(d, axis=1, keepdims=True)
            iota = jax.lax.broadcasted_iota(jnp.int32, d.shape, 1)
            idx = jnp.min(jnp.where(d == dmin, iota, N_CODES), axis=1)
            idx_ref[0, i * N_GROUPS + g, :] = idx
            onehot = (iota == idx[:, None]).astype(jnp.bfloat16)
            wb0 = wb0_ref[i, g]
            wb1 = wb1_ref[i, g]
            zg = (jax.lax.dot_general(
                onehot, wb0, (((1,), (0,)), ((), ())),
                preferred_element_type=jnp.float32)
                + jax.lax.dot_general(
                onehot, wb1, (((1,), (0,)), ((), ())),
                preferred_element_type=jnp.float32))
            zparts.append(zg)
        z = jnp.concatenate(zparts, axis=1)
        r = r - z
        qsum = qsum + z
        ss_list.append(jnp.sum(r * r))
    qout_ref[...] = qsum
    ss_vec = jnp.stack(ss_list)[:, None] * jnp.ones((1, 128), jnp.float32)

    @pl.when(step == 0)
    def _init():
        ss_ref[...] = jnp.zeros_like(ss_ref)

    ss_ref[...] += ss_vec


def _ce_body(w_ref, out_ref):
    w = w_ref[0]
    gmat = 3.0 * jax.lax.dot_general(
        w, w, (((1,), (1,)), ((), ())), precision=jax.lax.Precision.HIGHEST)
    mx = jnp.max(gmat, axis=1, keepdims=True)
    lse = jnp.log(jnp.sum(jnp.exp(gmat - mx), axis=1, keepdims=True)) + mx
    diag = 3.0 * jnp.sum(w * w, axis=1, keepdims=True)
    out_ref[0] = jnp.full((8, 128), jnp.mean(lse - diag), jnp.float32)


def kernel(xin, codebooks):
    xt = jnp.transpose(xin, (0, 2, 1)).reshape(N_TOK, VQ_DIM)
    wn = jnp.sum(codebooks ** 2, axis=-1).reshape(N_Q * N_GROUPS, N_CODES)

    qout_t, idx_raw, ss_raw = pl.pallas_call(
        _rvq_body,
        grid=(GRID,),
        in_specs=[
            pl.BlockSpec((TB, VQ_DIM), lambda s: (s, 0)),
            pl.BlockSpec((N_Q, N_GROUPS, N_CODES, CODE_DIM),
                         lambda s: (0, 0, 0, 0)),
            pl.BlockSpec((N_Q * N_GROUPS, N_CODES), lambda s: (0, 0)),
        ],
        out_specs=[
            pl.BlockSpec((TB, VQ_DIM), lambda s: (s, 0)),
            pl.BlockSpec((1, N_Q * N_GROUPS, TB), lambda s: (s, 0, 0)),
            pl.BlockSpec((N_Q, 128), lambda s: (0, 0)),
        ],
        out_shape=[
            jax.ShapeDtypeStruct((N_TOK, VQ_DIM), jnp.float32),
            jax.ShapeDtypeStruct((GRID, N_Q * N_GROUPS, TB), jnp.int32),
            jax.ShapeDtypeStruct((N_Q, 128), jnp.float32),
        ],
        scratch_shapes=[
            pltpu.VMEM((N_Q, N_GROUPS, N_CODES, CODE_DIM), jnp.bfloat16),
            pltpu.VMEM((N_Q, N_GROUPS, N_CODES, CODE_DIM), jnp.bfloat16),
        ],
    )(xt, codebooks, wn)

    ce_out = pl.pallas_call(
        _ce_body,
        grid=(N_GROUPS,),
        in_specs=[pl.BlockSpec((1, N_CODES, CODE_DIM), lambda g: (g, 0, 0))],
        out_specs=pl.BlockSpec((1, 8, 128), lambda g: (g, 0, 0)),
        out_shape=jax.ShapeDtypeStruct((N_GROUPS, 8, 128), jnp.float32),
    )(codebooks[0])

    quantized_out = jnp.transpose(qout_t.reshape(16, 2048, VQ_DIM), (0, 2, 1))
    indices = jnp.transpose(idx_raw, (1, 0, 2)).reshape(N_Q * N_GROUPS, N_TOK)

    numel = jnp.float32(16 * VQ_DIM * 2048)
    msq = ss_raw[:, 0] / numel
    ce = ce_out[:, 0, 0]
    e0 = (ce[0] + ce[1]) * 0.5
    e = jnp.concatenate([e0[None], jnp.zeros((N_Q - 1,), jnp.float32)])
    loss = jnp.mean(0.1 * e + 1.0 * msq + 0.25 * msq)
    return (quantized_out, loss, indices)


# speedup vs baseline: 2.2694x; 2.2694x over previous
import jax
import jax.numpy as jnp
from jax.experimental import pallas as pl
from jax.experimental.pallas import tpu as pltpu

N_Q = 8
N_GROUPS = 2
N_CODES = 1024
VQ_DIM = 512
CODE_DIM = VQ_DIM // N_GROUPS
TB = 512
N_TOK = 16 * 2048
GRID = N_TOK // TB


def _rvq_body(xt_ref, cb_ref, wn_ref, qout_ref, idx_ref, ss_ref):
    step = pl.program_id(0)
    x = xt_ref[...]
    r = x
    qsum = jnp.zeros_like(x)
    ss_list = []
    for i in range(N_Q):
        zparts = []
        for g in range(N_GROUPS):
            rg = r[:, g * CODE_DIM:(g + 1) * CODE_DIM]
            a = jnp.sum(rg * rg, axis=1, keepdims=True)
            w = cb_ref[i, g]
            m = jax.lax.dot_general(
                rg, w, (((1,), (1,)), ((), ())),
                precision=jax.lax.Precision.DEFAULT)
            b = wn_ref[i * N_GROUPS + g, :][None, :]
            d = (a + b) - 2.0 * m
            dmin = jnp.min(d, axis=1, keepdims=True)
            iota = jax.lax.broadcasted_iota(jnp.int32, d.shape, 1)
            idx = jnp.min(jnp.where(d == dmin, iota, N_CODES), axis=1)
            idx_ref[0, i * N_GROUPS + g, :] = idx
            onehot = (iota == idx[:, None]).astype(jnp.bfloat16)
            wb0 = w.astype(jnp.bfloat16)
            wb1 = (w - wb0.astype(jnp.float32)).astype(jnp.bfloat16)
            zg = (jax.lax.dot_general(
                onehot, wb0, (((1,), (0,)), ((), ())),
                preferred_element_type=jnp.float32)
                + jax.lax.dot_general(
                onehot, wb1, (((1,), (0,)), ((), ())),
                preferred_element_type=jnp.float32))
            zparts.append(zg)
        z = jnp.concatenate(zparts, axis=1)
        r = r - z
        qsum = qsum + z
        ss_list.append(jnp.sum(r * r))
    qout_ref[...] = qsum
    ss_vec = jnp.stack(ss_list)[:, None] * jnp.ones((1, 128), jnp.float32)

    @pl.when(step == 0)
    def _init():
        ss_ref[...] = jnp.zeros_like(ss_ref)

    ss_ref[...] += ss_vec


def _ce_body(w_ref, out_ref):
    w = w_ref[0]
    gmat = 3.0 * jax.lax.dot_general(
        w, w, (((1,), (1,)), ((), ())), precision=jax.lax.Precision.HIGHEST)
    mx = jnp.max(gmat, axis=1, keepdims=True)
    lse = jnp.log(jnp.sum(jnp.exp(gmat - mx), axis=1, keepdims=True)) + mx
    diag = 3.0 * jnp.sum(w * w, axis=1, keepdims=True)
    out_ref[0] = jnp.full((8, 128), jnp.mean(lse - diag), jnp.float32)


def kernel(xin, codebooks):
    xt = jnp.transpose(xin, (0, 2, 1)).reshape(N_TOK, VQ_DIM)
    wn = jnp.sum(codebooks ** 2, axis=-1).reshape(N_Q * N_GROUPS, N_CODES)

    qout_t, idx_raw, ss_raw = pl.pallas_call(
        _rvq_body,
        grid=(GRID,),
        in_specs=[
            pl.BlockSpec((TB, VQ_DIM), lambda s: (s, 0)),
            pl.BlockSpec((N_Q, N_GROUPS, N_CODES, CODE_DIM),
                         lambda s: (0, 0, 0, 0)),
            pl.BlockSpec((N_Q * N_GROUPS, N_CODES), lambda s: (0, 0)),
        ],
        out_specs=[
            pl.BlockSpec((TB, VQ_DIM), lambda s: (s, 0)),
            pl.BlockSpec((1, N_Q * N_GROUPS, TB), lambda s: (s, 0, 0)),
            pl.BlockSpec((N_Q, 128), lambda s: (0, 0)),
        ],
        out_shape=[
            jax.ShapeDtypeStruct((N_TOK, VQ_DIM), jnp.float32),
            jax.ShapeDtypeStruct((GRID, N_Q * N_GROUPS, TB), jnp.int32),
            jax.ShapeDtypeStruct((N_Q, 128), jnp.float32),
        ],
    )(xt, codebooks, wn)

    ce_out = pl.pallas_call(
        _ce_body,
        grid=(N_GROUPS,),
        in_specs=[pl.BlockSpec((1, N_CODES, CODE_DIM), lambda g: (g, 0, 0))],
        out_specs=pl.BlockSpec((1, 8, 128), lambda g: (g, 0, 0)),
        out_shape=jax.ShapeDtypeStruct((N_GROUPS, 8, 128), jnp.float32),
    )(codebooks[0])

    quantized_out = jnp.transpose(qout_t.reshape(16, 2048, VQ_DIM), (0, 2, 1))
    indices = jnp.transpose(idx_raw, (1, 0, 2)).reshape(N_Q * N_GROUPS, N_TOK)

    numel = jnp.float32(16 * VQ_DIM * 2048)
    msq = ss_raw[:, 0] / numel
    ce = ce_out[:, 0, 0]
    e0 = (ce[0] + ce[1]) * 0.5
    e = jnp.concatenate([e0[None], jnp.zeros((N_Q - 1,), jnp.float32)])
    loss = jnp.mean(0.1 * e + 1.0 * msq + 0.25 * msq)
    return (quantized_out, loss, indices)
